# Initial kernel scaffold; baseline (speedup 1.0000x reference)
import functools

import jax
import jax.numpy as jnp
from jax import lax
from jax.experimental import pallas as pl
from jax.experimental.pallas import tpu as pltpu
from jax.experimental.pallas import tpu_sc as plsc

B, S, C = 32, 8192, 128
K = 8
LANES = 16
NCG = C // LANES
ROWS = 256
NCHUNK = S // ROWS


def _kmax_body(x_hbm, out_hbm, buf, runner, outbuf):
    cid = lax.axis_index("c")
    sid = lax.axis_index("s")
    b = sid * 2 + cid

    neg_inf = jnp.full((LANES,), -jnp.inf, dtype=jnp.float32)
    for j in range(K):
        for cg in range(NCG):
            runner[j, pl.ds(cg * LANES, LANES)] = neg_inf

    def chunk_body(chunk, carry):
        pltpu.sync_copy(x_hbm.at[b, pl.ds(chunk * ROWS, ROWS), :], buf)
        for cg in range(NCG):
            cbase = cg * LANES

            def row_body(i, rcarry):
                v = buf[i, pl.ds(cbase, LANES)]
                out = []
                for j in range(K):
                    hi = jnp.maximum(rcarry[j], v)
                    v = jnp.minimum(rcarry[j], v)
                    out.append(hi)
                return tuple(out)

            init = tuple(runner[j, pl.ds(cbase, LANES)] for j in range(K))
            res = lax.fori_loop(0, ROWS, row_body, init)
            for j in range(K):
                runner[j, pl.ds(cbase, LANES)] = res[j]
        return carry

    lax.fori_loop(0, NCHUNK, chunk_body, 0)

    lane = lax.iota(jnp.int32, LANES)
    for cg in range(NCG):
        for j in range(K):
            v = runner[j, pl.ds(cg * LANES, LANES)]
            idx = lane * K + (cg * LANES * K + j)
            plsc.store_scatter(outbuf, [idx], v)
    pltpu.sync_copy(outbuf, out_hbm.at[b])


@jax.jit
def kernel(inputs):
    f = functools.partial(
        pl.kernel,
        mesh=plsc.VectorSubcoreMesh(core_axis_name="c", subcore_axis_name="s"),
        out_type=jax.ShapeDtypeStruct((B, C * K), jnp.float32),
        scratch_types=[
            pltpu.VMEM((ROWS, C), jnp.float32),
            pltpu.VMEM((K, C), jnp.float32),
            pltpu.VMEM((C * K,), jnp.float32),
        ],
    )(_kmax_body)
    return f(inputs)


# speedup vs baseline: 29.0571x; 29.0571x over previous
import functools

import jax
import jax.numpy as jnp
from jax import lax
from jax.experimental import pallas as pl
from jax.experimental.pallas import tpu as pltpu
from jax.experimental.pallas import tpu_sc as plsc

B, S, C = 32, 8192, 128
K = 8
LANES = 16
NCG = C // LANES
ROWS = 256
NCHUNK = S // ROWS


def _kmax_body(x_hbm, out_hbm, buf, runner, outbuf):
    cid = lax.axis_index("c")
    sid = lax.axis_index("s")
    b = sid * 2 + cid

    neg_inf = jnp.full((LANES,), -jnp.inf, dtype=jnp.float32)
    for j in range(K):
        for cg in range(NCG):
            runner[j, pl.ds(cg * LANES, LANES)] = neg_inf

    def chunk_body(chunk, carry):
        pltpu.sync_copy(x_hbm.at[b, pl.ds(chunk * ROWS, ROWS), :], buf)
        for cg in range(NCG):
            cbase = cg * LANES

            def row_body(i, rcarry):
                v = buf[i, pl.ds(cbase, LANES)]
                out = []
                for j in range(K):
                    hi = jnp.maximum(rcarry[j], v)
                    v = jnp.minimum(rcarry[j], v)
                    out.append(hi)
                return tuple(out)

            init = tuple(runner[j, pl.ds(cbase, LANES)] for j in range(K))
            res = lax.fori_loop(0, ROWS, row_body, init)
            for j in range(K):
                runner[j, pl.ds(cbase, LANES)] = res[j]
        return carry

    lax.fori_loop(0, NCHUNK, chunk_body, 0)

    lane = lax.iota(jnp.int32, LANES)
    for cg in range(NCG):
        for j in range(K):
            v = runner[j, pl.ds(cg * LANES, LANES)]
            idx = lane * K + (cg * LANES * K + j)
            plsc.store_scatter(outbuf, [idx], v)
    pltpu.sync_copy(outbuf, out_hbm.at[b])


@jax.jit
def kernel(inputs):
    f = functools.partial(
        pl.kernel,
        mesh=plsc.VectorSubcoreMesh(core_axis_name="c", subcore_axis_name="s"),
        compiler_params=pltpu.CompilerParams(needs_layout_passes=False),
        out_type=jax.ShapeDtypeStruct((B, C * K), jnp.float32),
        scratch_types=[
            pltpu.VMEM((ROWS, C), jnp.float32),
            pltpu.VMEM((K, C), jnp.float32),
            pltpu.VMEM((C * K,), jnp.float32),
        ],
    )(_kmax_body)
    return f(inputs)
